# Initial kernel scaffold; baseline (speedup 1.0000x reference)
import jax
import jax.numpy as jnp
from jax.experimental import pallas as pl


def kernel(query, key_matrix, val_matrix):
    raise NotImplementedError("write your pallas kernel here")


# speedup vs baseline: 1.7906x; 1.7906x over previous
import functools

import jax
import jax.numpy as jnp
from jax import lax
from jax.experimental import pallas as pl
from jax.experimental.pallas import tpu as pltpu
from jax.experimental.pallas import tpu_sc as plsc

_B = 4096
_N = 65536
_D = 256
_K = 8

_BQ = 256
_BK = 2048
_BIG_I = 2 ** 30


def _select_top8(values, ids, k_top):
    vs, is_ = [], []
    for _ in range(k_top):
        m = jnp.max(values, axis=1, keepdims=True)
        sel = jnp.where(values == m, ids, _BIG_I)
        mi = jnp.min(sel, axis=1, keepdims=True)
        vs.append(m)
        is_.append(mi)
        values = jnp.where(ids == mi, -jnp.inf, values)
    return jnp.concatenate(vs, axis=1), jnp.concatenate(is_, axis=1)


def _topk_kernel(q_ref, k_ref, w_ref, i_ref, vals, idxs):
    kk = pl.program_id(1)
    nk = pl.num_programs(1)

    @pl.when(kk == 0)
    def _init():
        vals[...] = jnp.full((_BQ, _K), -jnp.inf, jnp.float32)
        idxs[...] = jnp.full((_BQ, _K), _BIG_I, jnp.int32)

    q = q_ref[...]
    qn = q / jnp.clip(jnp.sqrt(jnp.sum(q * q, axis=1, keepdims=True)), 1e-12)
    kb = k_ref[...]
    kn = kb / jnp.clip(jnp.sqrt(jnp.sum(kb * kb, axis=1, keepdims=True)),
                       1e-12)
    s = lax.dot_general(qn, kn, (((1,), (1,)), ((), ())),
                        preferred_element_type=jnp.float32)
    colid = lax.broadcasted_iota(jnp.int32, (_BQ, _BK), 1) + kk * _BK

    tv, ti = _select_top8(s, colid, _K)

    cv = jnp.concatenate([vals[...], tv], axis=1)
    ci = jnp.concatenate([idxs[...], ti], axis=1)
    nv, ni = _select_top8(cv, ci, _K)
    vals[...] = nv
    idxs[...] = ni

    @pl.when(kk == nk - 1)
    def _emit():
        m = jnp.max(nv, axis=1, keepdims=True)
        e = jnp.exp(nv - m)
        w_ref[...] = e / jnp.sum(e, axis=1, keepdims=True)
        i_ref[...] = ni


def _topk_call(query, key_matrix):
    grid = (_B // _BQ, _N // _BK)
    return pl.pallas_call(
        _topk_kernel,
        grid=grid,
        in_specs=[
            pl.BlockSpec((_BQ, _D), lambda i, k: (i, 0)),
            pl.BlockSpec((_BK, _D), lambda i, k: (k, 0)),
        ],
        out_specs=[
            pl.BlockSpec((_BQ, _K), lambda i, k: (i, 0)),
            pl.BlockSpec((_BQ, _K), lambda i, k: (i, 0)),
        ],
        out_shape=[
            jax.ShapeDtypeStruct((_B, _K), jnp.float32),
            jax.ShapeDtypeStruct((_B, _K), jnp.int32),
        ],
        scratch_shapes=[
            pltpu.VMEM((_BQ, _K), jnp.float32),
            pltpu.VMEM((_BQ, _K), jnp.int32),
        ],
        compiler_params=pltpu.CompilerParams(
            dimension_semantics=("arbitrary", "arbitrary")),
    )(query, key_matrix)


_QW = 128
_QC = 16


def _gather_body(val_hbm, idx_hbm, w_hbm, out_hbm, idx_v, w_v, rows_v, out_v,
                 sem):
    wid = lax.axis_index("s") * 2 + lax.axis_index("c")

    def chunk_body(t, carry):
        base_q = wid * _QW + t * _QC
        pltpu.sync_copy(idx_hbm.at[pl.ds(base_q * _K, _QC * _K)], idx_v)
        pltpu.sync_copy(w_hbm.at[pl.ds(base_q * _K, _QC * _K), :], w_v)
        pltpu.async_copy(val_hbm.at[idx_v], rows_v, sem).wait()

        def q_body(qq, c2):
            wvs = [w_v[qq * _K + j, :] for j in range(_K)]
            for c in range(_D // 16):
                acc = wvs[0] * rows_v[qq * _K, pl.ds(c * 16, 16)]
                for j in range(1, _K):
                    acc = acc + wvs[j] * rows_v[qq * _K + j, pl.ds(c * 16, 16)]
                out_v[qq, pl.ds(c * 16, 16)] = acc
            return c2

        lax.fori_loop(0, _QC, q_body, 0)
        pltpu.sync_copy(out_v, out_hbm.at[pl.ds(base_q, _QC)])
        return carry

    lax.fori_loop(0, _QW // _QC, chunk_body, 0)


def _gather_call(val_matrix, idx_flat, w_flat):
    mesh = plsc.VectorSubcoreMesh(core_axis_name="c", subcore_axis_name="s")
    kfn = functools.partial(
        pl.kernel,
        mesh=mesh,
        out_type=jax.ShapeDtypeStruct((_B, _D), jnp.float32),
        scratch_types=[
            pltpu.VMEM((_QC * _K,), jnp.int32),
            pltpu.VMEM((_QC * _K, 16), jnp.float32),
            pltpu.VMEM((_QC * _K, _D), jnp.float32),
            pltpu.VMEM((_QC, _D), jnp.float32),
            pltpu.SemaphoreType.DMA,
        ],
    )(_gather_body)
    return kfn(val_matrix, idx_flat, w_flat)


def kernel(query, key_matrix, val_matrix):
    w, i = _topk_call(query, key_matrix)
    w_wide = jnp.broadcast_to(w.reshape(_B * _K, 1), (_B * _K, 16))
    return _gather_call(val_matrix, i.reshape(-1), w_wide)


# speedup vs baseline: 3.3221x; 1.8553x over previous
import functools

import jax
import jax.numpy as jnp
from jax import lax
from jax.experimental import pallas as pl
from jax.experimental.pallas import tpu as pltpu
from jax.experimental.pallas import tpu_sc as plsc

_B = 4096
_N = 65536
_D = 256
_K = 8

_BQ = 256
_BK = 2048
_CB = 128
_NB = _N // _CB
_NBT = _BK // _CB
_BIG_I = 2 ** 30


def _select_top8(values, ids, k_top):
    vs, is_ = [], []
    for _ in range(k_top):
        m = jnp.max(values, axis=1, keepdims=True)
        sel = jnp.where(values == m, ids, _BIG_I)
        mi = jnp.min(sel, axis=1, keepdims=True)
        vs.append(m)
        is_.append(mi)
        values = jnp.where(ids == mi, -jnp.inf, values)
    return jnp.concatenate(vs, axis=1), jnp.concatenate(is_, axis=1)


def _sim_kernel(q_ref, k_ref, sim_ref, bm_ref):
    q = q_ref[...]
    qn = q / jnp.clip(jnp.sqrt(jnp.sum(q * q, axis=1, keepdims=True)), 1e-12)
    kb = k_ref[...]
    kn = kb / jnp.clip(jnp.sqrt(jnp.sum(kb * kb, axis=1, keepdims=True)),
                       1e-12)
    s = lax.dot_general(qn, kn, (((1,), (1,)), ((), ())),
                        preferred_element_type=jnp.float32)
    sim_ref[...] = s
    bm = jnp.max(s.reshape(_BQ, _NBT, _CB), axis=2)
    bm_ref[...] = bm.reshape(1, _BQ, _NBT)


def _sim_call(query, key_matrix):
    grid = (_B // _BQ, _N // _BK)
    return pl.pallas_call(
        _sim_kernel,
        grid=grid,
        in_specs=[
            pl.BlockSpec((_BQ, _D), lambda i, k: (i, 0)),
            pl.BlockSpec((_BK, _D), lambda i, k: (k, 0)),
        ],
        out_specs=[
            pl.BlockSpec((_BQ, _BK), lambda i, k: (i, k)),
            pl.BlockSpec((1, _BQ, _NBT), lambda i, k: (k, i, 0)),
        ],
        out_shape=[
            jax.ShapeDtypeStruct((_B, _N), jnp.float32),
            jax.ShapeDtypeStruct((_N // _BK, _B, _NBT), jnp.float32),
        ],
        compiler_params=pltpu.CompilerParams(
            dimension_semantics=("parallel", "arbitrary")),
    )(query, key_matrix)


def _blocksel_kernel(bm_ref, bid_ref):
    bm = bm_ref[...]
    ids = lax.broadcasted_iota(jnp.int32, (_BQ, _NB), 1)
    _, bi = _select_top8(bm, ids, _K)
    bid_ref[...] = bi


def _blocksel_call(blockmax):
    return pl.pallas_call(
        _blocksel_kernel,
        grid=(_B // _BQ,),
        in_specs=[pl.BlockSpec((_BQ, _NB), lambda i: (i, 0))],
        out_specs=pl.BlockSpec((_BQ, _K), lambda i: (i, 0)),
        out_shape=jax.ShapeDtypeStruct((_B, _K), jnp.int32),
    )(blockmax)


_QW = _B // 32
_QC = 16


def _compact_body(sim_hbm, gidx_hbm, out_hbm, idx_v, rows_v, sem):
    wid = lax.axis_index("s") * 2 + lax.axis_index("c")

    def chunk_body(t, carry):
        base_q = wid * _QW + t * _QC
        pltpu.sync_copy(gidx_hbm.at[pl.ds(base_q * _K, _QC * _K)], idx_v)
        pltpu.async_copy(sim_hbm.at[idx_v], rows_v, sem).wait()
        pltpu.sync_copy(rows_v, out_hbm.at[pl.ds(base_q * _K, _QC * _K)])
        return carry

    lax.fori_loop(0, _QW // _QC, chunk_body, 0)


def _compact_call(sim_rows, gidx_flat):
    mesh = plsc.VectorSubcoreMesh(core_axis_name="c", subcore_axis_name="s")
    kfn = functools.partial(
        pl.kernel,
        mesh=mesh,
        out_type=jax.ShapeDtypeStruct((_B * _K, _CB), jnp.float32),
        scratch_types=[
            pltpu.VMEM((_QC * _K,), jnp.int32),
            pltpu.VMEM((_QC * _K, _CB), jnp.float32),
            pltpu.SemaphoreType.DMA,
        ],
    )(_compact_body)
    return kfn(sim_rows, gidx_flat)


def _final_kernel(cand_ref, bid_ref, w_ref, i_ref):
    c = cand_ref[...]
    bid = bid_ref[...]
    colid = (bid[:, :, None] * _CB
             + lax.broadcasted_iota(jnp.int32, (_BQ, _K, _CB), 2))
    colid = colid.reshape(_BQ, _K * _CB)
    nv, ni = _select_top8(c, colid, _K)
    m = jnp.max(nv, axis=1, keepdims=True)
    e = jnp.exp(nv - m)
    w_ref[...] = e / jnp.sum(e, axis=1, keepdims=True)
    i_ref[...] = ni


def _final_call(cand, blockid):
    return pl.pallas_call(
        _final_kernel,
        grid=(_B // _BQ,),
        in_specs=[
            pl.BlockSpec((_BQ, _K * _CB), lambda i: (i, 0)),
            pl.BlockSpec((_BQ, _K), lambda i: (i, 0)),
        ],
        out_specs=[
            pl.BlockSpec((_BQ, _K), lambda i: (i, 0)),
            pl.BlockSpec((_BQ, _K), lambda i: (i, 0)),
        ],
        out_shape=[
            jax.ShapeDtypeStruct((_B, _K), jnp.float32),
            jax.ShapeDtypeStruct((_B, _K), jnp.int32),
        ],
    )(cand, blockid)


def _gather_body(val_hbm, idx_hbm, w_hbm, out_hbm, idx_v, w_v, rows_v, out_v,
                 sem):
    wid = lax.axis_index("s") * 2 + lax.axis_index("c")

    def chunk_body(t, carry):
        base_q = wid * _QW + t * _QC
        pltpu.sync_copy(idx_hbm.at[pl.ds(base_q * _K, _QC * _K)], idx_v)
        pltpu.sync_copy(w_hbm.at[pl.ds(base_q * _K, _QC * _K), :], w_v)
        pltpu.async_copy(val_hbm.at[idx_v], rows_v, sem).wait()

        def q_body(qq, c2):
            wvs = [w_v[qq * _K + j, :] for j in range(_K)]
            for c in range(_D // 16):
                acc = wvs[0] * rows_v[qq * _K, pl.ds(c * 16, 16)]
                for j in range(1, _K):
                    acc = acc + wvs[j] * rows_v[qq * _K + j, pl.ds(c * 16, 16)]
                out_v[qq, pl.ds(c * 16, 16)] = acc
            return c2

        lax.fori_loop(0, _QC, q_body, 0)
        pltpu.sync_copy(out_v, out_hbm.at[pl.ds(base_q, _QC)])
        return carry

    lax.fori_loop(0, _QW // _QC, chunk_body, 0)


def _gather_call(val_matrix, idx_flat, w_wide):
    mesh = plsc.VectorSubcoreMesh(core_axis_name="c", subcore_axis_name="s")
    kfn = functools.partial(
        pl.kernel,
        mesh=mesh,
        out_type=jax.ShapeDtypeStruct((_B, _D), jnp.float32),
        scratch_types=[
            pltpu.VMEM((_QC * _K,), jnp.int32),
            pltpu.VMEM((_QC * _K, 16), jnp.float32),
            pltpu.VMEM((_QC * _K, _D), jnp.float32),
            pltpu.VMEM((_QC, _D), jnp.float32),
            pltpu.SemaphoreType.DMA,
        ],
    )(_gather_body)
    return kfn(val_matrix, idx_flat, w_wide)


def kernel(query, key_matrix, val_matrix):
    sim, bm3 = _sim_call(query, key_matrix)
    blockmax = bm3.transpose(1, 0, 2).reshape(_B, _NB)
    blockid = _blocksel_call(blockmax)

    rowid = lax.broadcasted_iota(jnp.int32, (_B, _K), 0)
    gidx = (rowid * _NB + blockid).reshape(-1)
    cand = _compact_call(sim.reshape(_B * _NB, _CB), gidx)

    w, i = _final_call(cand.reshape(_B, _K * _CB), blockid)

    w_wide = jnp.broadcast_to(w.reshape(_B * _K, 1), (_B * _K, 16))
    return _gather_call(val_matrix, i.reshape(-1), w_wide)


# speedup vs baseline: 3.8890x; 1.1707x over previous
import functools

import jax
import jax.numpy as jnp
from jax import lax
from jax.experimental import pallas as pl
from jax.experimental.pallas import tpu as pltpu
from jax.experimental.pallas import tpu_sc as plsc

_B = 4096
_N = 65536
_D = 256
_K = 8

_BQ = 512
_BK = 2048
_CB = 128
_NB = _N // _CB
_NBT = _BK // _CB
_BIG_I = 2 ** 30


def _select_top8(values, ids, k_top):
    vs, is_ = [], []
    for _ in range(k_top):
        m = jnp.max(values, axis=1, keepdims=True)
        sel = jnp.where(values == m, ids, _BIG_I)
        mi = jnp.min(sel, axis=1, keepdims=True)
        vs.append(m)
        is_.append(mi)
        values = jnp.where(ids == mi, -jnp.inf, values)
    return jnp.concatenate(vs, axis=1), jnp.concatenate(is_, axis=1)


def _sim_kernel(q_ref, k_ref, sim_ref, bm_ref):
    q = q_ref[...]
    qn = q / jnp.clip(jnp.sqrt(jnp.sum(q * q, axis=1, keepdims=True)), 1e-12)
    kb = k_ref[...]
    kn = kb / jnp.clip(jnp.sqrt(jnp.sum(kb * kb, axis=1, keepdims=True)),
                       1e-12)
    s = lax.dot_general(qn, kn, (((1,), (1,)), ((), ())),
                        preferred_element_type=jnp.float32)
    sim_ref[...] = s
    bm = jnp.max(s.reshape(_BQ, _NBT, _CB), axis=2)
    bm_ref[...] = bm.reshape(1, _BQ, _NBT)


def _sim_call(query, key_matrix):
    grid = (_B // _BQ, _N // _BK)
    return pl.pallas_call(
        _sim_kernel,
        grid=grid,
        in_specs=[
            pl.BlockSpec((_BQ, _D), lambda i, k: (i, 0)),
            pl.BlockSpec((_BK, _D), lambda i, k: (k, 0)),
        ],
        out_specs=[
            pl.BlockSpec((_BQ, _BK), lambda i, k: (i, k)),
            pl.BlockSpec((1, _BQ, _NBT), lambda i, k: (k, i, 0)),
        ],
        out_shape=[
            jax.ShapeDtypeStruct((_B, _N), jnp.float32),
            jax.ShapeDtypeStruct((_N // _BK, _B, _NBT), jnp.float32),
        ],
        compiler_params=pltpu.CompilerParams(
            dimension_semantics=("parallel", "arbitrary")),
    )(query, key_matrix)


def _blocksel_kernel(bm_ref, bid_ref):
    bm = bm_ref[...]
    ids = lax.broadcasted_iota(jnp.int32, (_BQ, _NB), 1)
    _, bi = _select_top8(bm, ids, _K)
    bid_ref[...] = bi


def _blocksel_call(blockmax):
    return pl.pallas_call(
        _blocksel_kernel,
        grid=(_B // _BQ,),
        in_specs=[pl.BlockSpec((_BQ, _NB), lambda i: (i, 0))],
        out_specs=pl.BlockSpec((_BQ, _K), lambda i: (i, 0)),
        out_shape=jax.ShapeDtypeStruct((_B, _K), jnp.int32),
    )(blockmax)


_QW = _B // 32
_QC = 16


def _compact_body(sim_hbm, gidx_hbm, out_hbm, idx_v, rows_v, sem):
    wid = lax.axis_index("s") * 2 + lax.axis_index("c")

    def chunk_body(t, carry):
        base_q = wid * _QW + t * _QC
        pltpu.sync_copy(gidx_hbm.at[pl.ds(base_q * _K, _QC * _K)], idx_v)
        pltpu.async_copy(sim_hbm.at[idx_v], rows_v, sem).wait()
        pltpu.sync_copy(rows_v, out_hbm.at[pl.ds(base_q * _K, _QC * _K)])
        return carry

    lax.fori_loop(0, _QW // _QC, chunk_body, 0)


def _compact_call(sim_rows, gidx_flat):
    mesh = plsc.VectorSubcoreMesh(core_axis_name="c", subcore_axis_name="s")
    kfn = functools.partial(
        pl.kernel,
        mesh=mesh,
        out_type=jax.ShapeDtypeStruct((_B * _K, _CB), jnp.float32),
        scratch_types=[
            pltpu.VMEM((_QC * _K,), jnp.int32),
            pltpu.VMEM((_QC * _K, _CB), jnp.float32),
            pltpu.SemaphoreType.DMA,
        ],
    )(_compact_body)
    return kfn(sim_rows, gidx_flat)


def _final_kernel(cand_ref, bid_ref, w_ref, i_ref):
    c = cand_ref[...]
    bid = bid_ref[...]
    colid = (bid[:, :, None] * _CB
             + lax.broadcasted_iota(jnp.int32, (_BQ, _K, _CB), 2))
    colid = colid.reshape(_BQ, _K * _CB)
    nv, ni = _select_top8(c, colid, _K)
    m = jnp.max(nv, axis=1, keepdims=True)
    e = jnp.exp(nv - m)
    w_ref[...] = e / jnp.sum(e, axis=1, keepdims=True)
    i_ref[...] = ni


def _final_call(cand, blockid):
    return pl.pallas_call(
        _final_kernel,
        grid=(_B // _BQ,),
        in_specs=[
            pl.BlockSpec((_BQ, _K * _CB), lambda i: (i, 0)),
            pl.BlockSpec((_BQ, _K), lambda i: (i, 0)),
        ],
        out_specs=[
            pl.BlockSpec((_BQ, _K), lambda i: (i, 0)),
            pl.BlockSpec((_BQ, _K), lambda i: (i, 0)),
        ],
        out_shape=[
            jax.ShapeDtypeStruct((_B, _K), jnp.float32),
            jax.ShapeDtypeStruct((_B, _K), jnp.int32),
        ],
    )(cand, blockid)


def _gather_body(val_hbm, idx_hbm, w_hbm, out_hbm, idx_v, w_v, rows_v, out_v,
                 sem):
    wid = lax.axis_index("s") * 2 + lax.axis_index("c")

    def chunk_body(t, carry):
        base_q = wid * _QW + t * _QC
        pltpu.sync_copy(idx_hbm.at[pl.ds(base_q * _K, _QC * _K)], idx_v)
        pltpu.sync_copy(w_hbm.at[pl.ds(base_q * _K, _QC * _K), :], w_v)
        pltpu.async_copy(val_hbm.at[idx_v], rows_v, sem).wait()

        def q_body(qq, c2):
            wvs = [w_v[qq * _K + j, :] for j in range(_K)]
            for c in range(_D // 16):
                acc = wvs[0] * rows_v[qq * _K, pl.ds(c * 16, 16)]
                for j in range(1, _K):
                    acc = acc + wvs[j] * rows_v[qq * _K + j, pl.ds(c * 16, 16)]
                out_v[qq, pl.ds(c * 16, 16)] = acc
            return c2

        lax.fori_loop(0, _QC, q_body, 0)
        pltpu.sync_copy(out_v, out_hbm.at[pl.ds(base_q, _QC)])
        return carry

    lax.fori_loop(0, _QW // _QC, chunk_body, 0)


def _gather_call(val_matrix, idx_flat, w_wide):
    mesh = plsc.VectorSubcoreMesh(core_axis_name="c", subcore_axis_name="s")
    kfn = functools.partial(
        pl.kernel,
        mesh=mesh,
        out_type=jax.ShapeDtypeStruct((_B, _D), jnp.float32),
        scratch_types=[
            pltpu.VMEM((_QC * _K,), jnp.int32),
            pltpu.VMEM((_QC * _K, 16), jnp.float32),
            pltpu.VMEM((_QC * _K, _D), jnp.float32),
            pltpu.VMEM((_QC, _D), jnp.float32),
            pltpu.SemaphoreType.DMA,
        ],
    )(_gather_body)
    return kfn(val_matrix, idx_flat, w_wide)


def kernel(query, key_matrix, val_matrix):
    sim, bm3 = _sim_call(query, key_matrix)
    blockmax = bm3.transpose(1, 0, 2).reshape(_B, _NB)
    blockid = _blocksel_call(blockmax)

    rowid = lax.broadcasted_iota(jnp.int32, (_B, _K), 0)
    gidx = (rowid * _NB + blockid).reshape(-1)
    cand = _compact_call(sim.reshape(_B * _NB, _CB), gidx)

    w, i = _final_call(cand.reshape(_B, _K * _CB), blockid)

    w_wide = jnp.broadcast_to(w.reshape(_B * _K, 1), (_B * _K, 16))
    return _gather_call(val_matrix, i.reshape(-1), w_wide)


# speedup vs baseline: 4.0437x; 1.0398x over previous
import functools

import jax
import jax.numpy as jnp
from jax import lax
from jax.experimental import pallas as pl
from jax.experimental.pallas import tpu as pltpu
from jax.experimental.pallas import tpu_sc as plsc

_B = 4096
_N = 65536
_D = 256
_K = 8

_BQ = 1024
_BK = 2048
_CB = 128
_NB = _N // _CB
_NBT = _BK // _CB
_BIG_I = 2 ** 30


def _select_top8(values, ids, k_top):
    vs, is_ = [], []
    for _ in range(k_top):
        m = jnp.max(values, axis=1, keepdims=True)
        sel = jnp.where(values == m, ids, _BIG_I)
        mi = jnp.min(sel, axis=1, keepdims=True)
        vs.append(m)
        is_.append(mi)
        values = jnp.where(ids == mi, -jnp.inf, values)
    return jnp.concatenate(vs, axis=1), jnp.concatenate(is_, axis=1)


def _sim_kernel(q_ref, k_ref, sim_ref, bm_ref):
    q = q_ref[...]
    qn = q / jnp.clip(jnp.sqrt(jnp.sum(q * q, axis=1, keepdims=True)), 1e-12)
    kb = k_ref[...]
    kn = kb / jnp.clip(jnp.sqrt(jnp.sum(kb * kb, axis=1, keepdims=True)),
                       1e-12)
    s = lax.dot_general(qn, kn, (((1,), (1,)), ((), ())),
                        preferred_element_type=jnp.float32)
    sim_ref[...] = s
    bm = jnp.max(s.reshape(_BQ, _NBT, _CB), axis=2)
    bm_ref[...] = bm.reshape(1, _BQ, _NBT)


def _sim_call(query, key_matrix):
    grid = (_B // _BQ, _N // _BK)
    return pl.pallas_call(
        _sim_kernel,
        grid=grid,
        in_specs=[
            pl.BlockSpec((_BQ, _D), lambda i, k: (i, 0)),
            pl.BlockSpec((_BK, _D), lambda i, k: (k, 0)),
        ],
        out_specs=[
            pl.BlockSpec((_BQ, _BK), lambda i, k: (i, k)),
            pl.BlockSpec((1, _BQ, _NBT), lambda i, k: (k, i, 0)),
        ],
        out_shape=[
            jax.ShapeDtypeStruct((_B, _N), jnp.float32),
            jax.ShapeDtypeStruct((_N // _BK, _B, _NBT), jnp.float32),
        ],
        compiler_params=pltpu.CompilerParams(
            dimension_semantics=("parallel", "arbitrary")),
    )(query, key_matrix)


def _blocksel_kernel(bm_ref, bid_ref):
    bm = bm_ref[...]
    ids = lax.broadcasted_iota(jnp.int32, (_BQ, _NB), 1)
    _, bi = _select_top8(bm, ids, _K)
    bid_ref[...] = bi


def _blocksel_call(blockmax):
    return pl.pallas_call(
        _blocksel_kernel,
        grid=(_B // _BQ,),
        in_specs=[pl.BlockSpec((_BQ, _NB), lambda i: (i, 0))],
        out_specs=pl.BlockSpec((_BQ, _K), lambda i: (i, 0)),
        out_shape=jax.ShapeDtypeStruct((_B, _K), jnp.int32),
    )(blockmax)


_QW = _B // 32
_QC = 16


def _compact_body(sim_hbm, gidx_hbm, out_hbm, idx_v, rows_v, sem):
    wid = lax.axis_index("s") * 2 + lax.axis_index("c")

    def chunk_body(t, carry):
        base_q = wid * _QW + t * _QC
        pltpu.sync_copy(gidx_hbm.at[pl.ds(base_q * _K, _QC * _K)], idx_v)
        pltpu.async_copy(sim_hbm.at[idx_v], rows_v, sem).wait()
        pltpu.sync_copy(rows_v, out_hbm.at[pl.ds(base_q * _K, _QC * _K)])
        return carry

    lax.fori_loop(0, _QW // _QC, chunk_body, 0)


def _compact_call(sim_rows, gidx_flat):
    mesh = plsc.VectorSubcoreMesh(core_axis_name="c", subcore_axis_name="s")
    kfn = functools.partial(
        pl.kernel,
        mesh=mesh,
        out_type=jax.ShapeDtypeStruct((_B * _K, _CB), jnp.float32),
        scratch_types=[
            pltpu.VMEM((_QC * _K,), jnp.int32),
            pltpu.VMEM((_QC * _K, _CB), jnp.float32),
            pltpu.SemaphoreType.DMA,
        ],
    )(_compact_body)
    return kfn(sim_rows, gidx_flat)


def _final_kernel(cand_ref, bid_ref, w_ref, i_ref):
    c = cand_ref[...]
    bid = bid_ref[...]
    colid = (bid[:, :, None] * _CB
             + lax.broadcasted_iota(jnp.int32, (_BQ, _K, _CB), 2))
    colid = colid.reshape(_BQ, _K * _CB)
    nv, ni = _select_top8(c, colid, _K)
    m = jnp.max(nv, axis=1, keepdims=True)
    e = jnp.exp(nv - m)
    w_ref[...] = e / jnp.sum(e, axis=1, keepdims=True)
    i_ref[...] = ni


def _final_call(cand, blockid):
    return pl.pallas_call(
        _final_kernel,
        grid=(_B // _BQ,),
        in_specs=[
            pl.BlockSpec((_BQ, _K * _CB), lambda i: (i, 0)),
            pl.BlockSpec((_BQ, _K), lambda i: (i, 0)),
        ],
        out_specs=[
            pl.BlockSpec((_BQ, _K), lambda i: (i, 0)),
            pl.BlockSpec((_BQ, _K), lambda i: (i, 0)),
        ],
        out_shape=[
            jax.ShapeDtypeStruct((_B, _K), jnp.float32),
            jax.ShapeDtypeStruct((_B, _K), jnp.int32),
        ],
    )(cand, blockid)


def _gather_body(val_hbm, idx_hbm, w_hbm, out_hbm, idx_v, w_v, rows_v, out_v,
                 sem):
    wid = lax.axis_index("s") * 2 + lax.axis_index("c")

    def chunk_body(t, carry):
        base_q = wid * _QW + t * _QC
        pltpu.sync_copy(idx_hbm.at[pl.ds(base_q * _K, _QC * _K)], idx_v)
        pltpu.sync_copy(w_hbm.at[pl.ds(base_q * _K, _QC * _K), :], w_v)
        pltpu.async_copy(val_hbm.at[idx_v], rows_v, sem).wait()

        def q_body(qq, c2):
            wvs = [w_v[qq * _K + j, :] for j in range(_K)]
            for c in range(_D // 16):
                acc = wvs[0] * rows_v[qq * _K, pl.ds(c * 16, 16)]
                for j in range(1, _K):
                    acc = acc + wvs[j] * rows_v[qq * _K + j, pl.ds(c * 16, 16)]
                out_v[qq, pl.ds(c * 16, 16)] = acc
            return c2

        lax.fori_loop(0, _QC, q_body, 0)
        pltpu.sync_copy(out_v, out_hbm.at[pl.ds(base_q, _QC)])
        return carry

    lax.fori_loop(0, _QW // _QC, chunk_body, 0)


def _gather_call(val_matrix, idx_flat, w_wide):
    mesh = plsc.VectorSubcoreMesh(core_axis_name="c", subcore_axis_name="s")
    kfn = functools.partial(
        pl.kernel,
        mesh=mesh,
        out_type=jax.ShapeDtypeStruct((_B, _D), jnp.float32),
        scratch_types=[
            pltpu.VMEM((_QC * _K,), jnp.int32),
            pltpu.VMEM((_QC * _K, 16), jnp.float32),
            pltpu.VMEM((_QC * _K, _D), jnp.float32),
            pltpu.VMEM((_QC, _D), jnp.float32),
            pltpu.SemaphoreType.DMA,
        ],
    )(_gather_body)
    return kfn(val_matrix, idx_flat, w_wide)


def kernel(query, key_matrix, val_matrix):
    sim, bm3 = _sim_call(query, key_matrix)
    blockmax = bm3.transpose(1, 0, 2).reshape(_B, _NB)
    blockid = _blocksel_call(blockmax)

    rowid = lax.broadcasted_iota(jnp.int32, (_B, _K), 0)
    gidx = (rowid * _NB + blockid).reshape(-1)
    cand = _compact_call(sim.reshape(_B * _NB, _CB), gidx)

    w, i = _final_call(cand.reshape(_B, _K * _CB), blockid)

    w_wide = jnp.broadcast_to(w.reshape(_B * _K, 1), (_B * _K, 16))
    return _gather_call(val_matrix, i.reshape(-1), w_wide)
